# Initial kernel scaffold; baseline (speedup 1.0000x reference)
import jax
import jax.numpy as jnp
from jax.experimental import pallas as pl


def kernel(batch, m_topic, g_topic, m_emb, g_emb, t_emb, W1, b1, W2, b2, W3, b3):
    raise NotImplementedError("write your pallas kernel here")


# speedup vs baseline: 9.4248x; 9.4248x over previous
import functools

import jax
import jax.numpy as jnp
from jax import lax
from jax.experimental import pallas as pl
from jax.experimental.pallas import tpu as pltpu
from jax.experimental.pallas import tpu_sc as plsc

B = 4096
E = 128
H = 1024
N_TOPIC = 1000
NT_PAD = 1024
HIST = 50
HIST_PAD = 64

_info = plsc.get_sparse_core_info()
NC, NS = _info.num_cores, _info.num_subcores
NW = NC * NS
ROWS_PER_W = B // NW
CH = 64
N_CHUNK = ROWS_PER_W // CH


def _sc_body(b0, b1, b2, mt, gt, m_emb, g_emb,
             org, gg, mem, cm, cg,
             idx_v, rows_v, ids_v, counts_v, sem):
    wid = lax.axis_index("s") * NC + lax.axis_index("c")
    base = wid * ROWS_PER_W

    for idx_hbm, tab, out in ((b0, m_emb, org), (b1, g_emb, gg), (b2, m_emb, mem)):
        pltpu.sync_copy(idx_hbm.at[pl.ds(base, ROWS_PER_W)], idx_v)
        pltpu.async_copy(tab.at[idx_v], rows_v, sem).wait()
        pltpu.sync_copy(rows_v, out.at[pl.ds(base, ROWS_PER_W)])

    zeros16 = jnp.zeros((16,), jnp.float32)
    ones16 = jnp.ones((16,), jnp.float32)
    lane = lax.iota(jnp.int32, 16)
    full_mask = lane < 16
    tail_mask = lane < (HIST - 3 * 16)

    def zero_body(i, _):
        counts_v[pl.ds(pl.multiple_of(i * 16, 16), 16)] = zeros16
        return 0

    def scatter_body(r, _):
        rbase = pl.multiple_of(r * HIST_PAD, 16)
        off0 = pl.multiple_of(r * NT_PAD, 16)
        for j in range(4):
            idvec = ids_v[pl.ds(rbase + j * 16, 16)]
            off = idvec + off0
            msk = full_mask if j < 3 else tail_mask
            plsc.addupdate_scatter(counts_v, [off], ones16, mask=msk)
        return 0

    for tab, outf in ((mt, cm), (gt, cg)):
        for c in range(N_CHUNK):
            row0 = base + c * CH
            pltpu.sync_copy(tab.at[pl.ds(row0 * HIST_PAD, CH * HIST_PAD)], ids_v)
            lax.fori_loop(0, CH * NT_PAD // 16, zero_body, 0)
            lax.fori_loop(0, CH, scatter_body, 0)
            pltpu.sync_copy(counts_v, outf.at[pl.ds(row0 * NT_PAD, CH * NT_PAD)])


@functools.partial(jax.jit, static_argnames=())
def _sc_call(b0, b1, b2, mt_flat, gt_flat, m_emb, g_emb):
    kern = pl.kernel(
        _sc_body,
        out_type=[
            jax.ShapeDtypeStruct((B, E), jnp.float32),
            jax.ShapeDtypeStruct((B, E), jnp.float32),
            jax.ShapeDtypeStruct((B, E), jnp.float32),
            jax.ShapeDtypeStruct((B * NT_PAD,), jnp.float32),
            jax.ShapeDtypeStruct((B * NT_PAD,), jnp.float32),
        ],
        mesh=plsc.VectorSubcoreMesh(core_axis_name="c", subcore_axis_name="s"),
        compiler_params=pltpu.CompilerParams(needs_layout_passes=False),
        scratch_types=[
            pltpu.VMEM((ROWS_PER_W,), jnp.int32),
            pltpu.VMEM((ROWS_PER_W, E), jnp.float32),
            pltpu.VMEM((CH * HIST_PAD,), jnp.int32),
            pltpu.VMEM((CH * NT_PAD,), jnp.float32),
            pltpu.SemaphoreType.DMA,
        ],
    )
    return kern(b0, b1, b2, mt_flat, gt_flat, m_emb, g_emb)


def _tc_body(org_ref, gg_ref, mem_ref, cm_ref, cg_ref, temb_ref,
             w1_ref, b1_ref, w2_ref, b2_ref, w3_ref, b3_ref, out_ref):
    te = temb_ref[...]
    pm = jnp.dot(cm_ref[...], te, preferred_element_type=jnp.float32) / 50.0
    pg = jnp.dot(cg_ref[...], te, preferred_element_type=jnp.float32) / 50.0
    x = jnp.concatenate(
        [org_ref[...], gg_ref[...], mem_ref[...], pm, pg], axis=1)
    h = jnp.dot(x, w1_ref[...], preferred_element_type=jnp.float32) + b1_ref[...]
    h = jnp.maximum(h, 0.0)
    h = jnp.dot(h, w2_ref[...], preferred_element_type=jnp.float32) + b2_ref[...]
    h = jnp.maximum(h, 0.0)
    out_ref[...] = (jnp.dot(h, w3_ref[...], preferred_element_type=jnp.float32)
                    + b3_ref[...])


def _tc_call(org, gg, mem, cm, cg, t_emb_pad, W1, b1, W2, b2, W3, b3):
    BLK = 512
    grid = (B // BLK,)
    n_out = W3.shape[1]

    def row_spec(width):
        return pl.BlockSpec((BLK, width), lambda i: (i, 0))

    def full_spec(shape):
        return pl.BlockSpec(shape, lambda i: tuple(0 for _ in shape))

    return pl.pallas_call(
        _tc_body,
        grid=grid,
        in_specs=[
            row_spec(E), row_spec(E), row_spec(E),
            row_spec(NT_PAD), row_spec(NT_PAD),
            full_spec((NT_PAD, E)),
            full_spec(W1.shape), full_spec((1, H)),
            full_spec(W2.shape), full_spec((1, H)),
            full_spec(W3.shape), full_spec((1, n_out)),
        ],
        out_specs=row_spec(n_out),
        out_shape=jax.ShapeDtypeStruct((B, n_out), jnp.float32),
        compiler_params=pltpu.CompilerParams(
            dimension_semantics=("parallel",),
        ),
    )(org, gg, mem, cm, cg, t_emb_pad, W1, b1, W2, b2, W3, b3)


def kernel(batch, m_topic, g_topic, m_emb, g_emb, t_emb, W1, b1, W2, b2, W3, b3):
    b0 = batch[:, 0].astype(jnp.int32)
    b1i = batch[:, 1].astype(jnp.int32)
    b2i = batch[:, 2].astype(jnp.int32)
    mt = jnp.pad(m_topic.astype(jnp.int32), ((0, 0), (0, HIST_PAD - HIST)))
    gt = jnp.pad(g_topic.astype(jnp.int32), ((0, 0), (0, HIST_PAD - HIST)))
    t_emb_pad = jnp.pad(t_emb, ((0, NT_PAD - N_TOPIC), (0, 0)))

    org, gg, mem, cm_flat, cg_flat = _sc_call(
        b0, b1i, b2i, mt.reshape(-1), gt.reshape(-1), m_emb, g_emb)
    cm = cm_flat.reshape(B, NT_PAD)
    cg = cg_flat.reshape(B, NT_PAD)

    logits = _tc_call(org, gg, mem, cm, cg, t_emb_pad,
                      W1, b1.reshape(1, H), W2, b2.reshape(1, H),
                      W3, b3.reshape(1, -1))
    return (logits,)


# speedup vs baseline: 16.5216x; 1.7530x over previous
import functools

import jax
import jax.numpy as jnp
from jax import lax
from jax.experimental import pallas as pl
from jax.experimental.pallas import tpu as pltpu
from jax.experimental.pallas import tpu_sc as plsc

B = 4096
E = 128
H = 1024
N_TOPIC = 1000
NT_PAD = 1024
HIST = 50
HIST_PAD = 64

_info = plsc.get_sparse_core_info()
NC, NS = _info.num_cores, _info.num_subcores
NW = NC * NS
ROWS_PER_W = B // NW
CH = 64
N_CHUNK = ROWS_PER_W // CH


def _sc_body(b0, b1, b2, mt, gt, m_emb, g_emb,
             org, gg, mem, cm, cg,
             idx_v, rows_v, ids_v, counts_v, sem):
    wid = lax.axis_index("s") * NC + lax.axis_index("c")
    base = wid * ROWS_PER_W

    for idx_hbm, tab, out in ((b0, m_emb, org), (b1, g_emb, gg), (b2, m_emb, mem)):
        pltpu.sync_copy(idx_hbm.at[pl.ds(base, ROWS_PER_W)], idx_v)
        pltpu.async_copy(tab.at[idx_v], rows_v, sem).wait()
        pltpu.sync_copy(rows_v, out.at[pl.ds(base, ROWS_PER_W)])

    zeros16 = jnp.zeros((16,), jnp.float32)
    ones16 = jnp.ones((16,), jnp.float32)
    lane = lax.iota(jnp.int32, 16)
    full_mask = lane < 16
    tail_mask = lane < (HIST - 3 * 16)

    def zero_row(r, _):
        for j in range(NT_PAD // 16):
            counts_v[r, pl.ds(j * 16, 16)] = zeros16
        return 0

    def make_scatter(vals):
        def scatter_body(r, _):
            rbase = pl.multiple_of(r * HIST_PAD, 16)
            rvec = jnp.full((16,), r, jnp.int32)
            for j in range(4):
                idvec = ids_v[pl.ds(rbase + j * 16, 16)]
                msk = full_mask if j < 3 else tail_mask
                if vals is None:
                    plsc.store_scatter(counts_v, [rvec, idvec], zeros16, mask=msk)
                else:
                    plsc.addupdate_scatter(counts_v, [rvec, idvec], vals, mask=msk)
            return 0
        return scatter_body

    scatter_add = make_scatter(ones16)
    scatter_zero = make_scatter(None)

    lax.fori_loop(0, CH, zero_row, 0)
    work = [(tab, outf, c) for tab, outf in ((mt, cm), (gt, cg))
            for c in range(N_CHUNK)]
    for k, (tab, outf, c) in enumerate(work):
        row0 = base + c * CH
        pltpu.sync_copy(tab.at[pl.ds(row0 * HIST_PAD, CH * HIST_PAD)], ids_v)
        lax.fori_loop(0, CH, scatter_add, 0)
        pltpu.sync_copy(counts_v, outf.at[pl.ds(row0, CH)])
        if k + 1 < len(work):
            lax.fori_loop(0, CH, scatter_zero, 0)


@functools.partial(jax.jit, static_argnames=())
def _sc_call(b0, b1, b2, mt_flat, gt_flat, m_emb, g_emb):
    kern = pl.kernel(
        _sc_body,
        out_type=[
            jax.ShapeDtypeStruct((B, E), jnp.float32),
            jax.ShapeDtypeStruct((B, E), jnp.float32),
            jax.ShapeDtypeStruct((B, E), jnp.float32),
            jax.ShapeDtypeStruct((B, NT_PAD), jnp.float32),
            jax.ShapeDtypeStruct((B, NT_PAD), jnp.float32),
        ],
        mesh=plsc.VectorSubcoreMesh(core_axis_name="c", subcore_axis_name="s"),
        compiler_params=pltpu.CompilerParams(needs_layout_passes=False),
        scratch_types=[
            pltpu.VMEM((ROWS_PER_W,), jnp.int32),
            pltpu.VMEM((ROWS_PER_W, E), jnp.float32),
            pltpu.VMEM((CH * HIST_PAD,), jnp.int32),
            pltpu.VMEM((CH, NT_PAD), jnp.float32),
            pltpu.SemaphoreType.DMA,
        ],
    )
    return kern(b0, b1, b2, mt_flat, gt_flat, m_emb, g_emb)


def _tc_body(org_ref, gg_ref, mem_ref, cm_ref, cg_ref, temb_ref,
             w1_ref, b1_ref, w2_ref, b2_ref, w3_ref, b3_ref, out_ref):
    te = temb_ref[...]
    pm = jnp.dot(cm_ref[...], te, preferred_element_type=jnp.float32) / 50.0
    pg = jnp.dot(cg_ref[...], te, preferred_element_type=jnp.float32) / 50.0
    x = jnp.concatenate(
        [org_ref[...], gg_ref[...], mem_ref[...], pm, pg], axis=1)
    h = jnp.dot(x, w1_ref[...], preferred_element_type=jnp.float32) + b1_ref[...]
    h = jnp.maximum(h, 0.0)
    h = jnp.dot(h, w2_ref[...], preferred_element_type=jnp.float32) + b2_ref[...]
    h = jnp.maximum(h, 0.0)
    out_ref[...] = (jnp.dot(h, w3_ref[...], preferred_element_type=jnp.float32)
                    + b3_ref[...])


def _tc_call(org, gg, mem, cm, cg, t_emb_pad, W1, b1, W2, b2, W3, b3):
    BLK = 512
    grid = (B // BLK,)
    n_out = W3.shape[1]

    def row_spec(width):
        return pl.BlockSpec((BLK, width), lambda i: (i, 0))

    def full_spec(shape):
        return pl.BlockSpec(shape, lambda i: tuple(0 for _ in shape))

    return pl.pallas_call(
        _tc_body,
        grid=grid,
        in_specs=[
            row_spec(E), row_spec(E), row_spec(E),
            row_spec(NT_PAD), row_spec(NT_PAD),
            full_spec((NT_PAD, E)),
            full_spec(W1.shape), full_spec((1, H)),
            full_spec(W2.shape), full_spec((1, H)),
            full_spec(W3.shape), full_spec((1, n_out)),
        ],
        out_specs=row_spec(n_out),
        out_shape=jax.ShapeDtypeStruct((B, n_out), jnp.float32),
        compiler_params=pltpu.CompilerParams(
            dimension_semantics=("parallel",),
        ),
    )(org, gg, mem, cm, cg, t_emb_pad, W1, b1, W2, b2, W3, b3)


def kernel(batch, m_topic, g_topic, m_emb, g_emb, t_emb, W1, b1, W2, b2, W3, b3):
    b0 = batch[:, 0].astype(jnp.int32)
    b1i = batch[:, 1].astype(jnp.int32)
    b2i = batch[:, 2].astype(jnp.int32)
    mt = jnp.pad(m_topic.astype(jnp.int32), ((0, 0), (0, HIST_PAD - HIST)))
    gt = jnp.pad(g_topic.astype(jnp.int32), ((0, 0), (0, HIST_PAD - HIST)))
    t_emb_pad = jnp.pad(t_emb, ((0, NT_PAD - N_TOPIC), (0, 0)))

    org, gg, mem, cm, cg = _sc_call(
        b0, b1i, b2i, mt.reshape(-1), gt.reshape(-1), m_emb, g_emb)

    logits = _tc_call(org, gg, mem, cm, cg, t_emb_pad,
                      W1, b1.reshape(1, H), W2, b2.reshape(1, H),
                      W3, b3.reshape(1, -1))
    return (logits,)


# speedup vs baseline: 23.5964x; 1.4282x over previous
import functools

import jax
import jax.numpy as jnp
from jax import lax
from jax.experimental import pallas as pl
from jax.experimental.pallas import tpu as pltpu
from jax.experimental.pallas import tpu_sc as plsc

B = 4096
E = 128
H = 1024
N_TOPIC = 1000
NT_PAD = 1024
N_OUT = 1000
HIST = 50

_info = plsc.get_sparse_core_info()
NC, NS = _info.num_cores, _info.num_subcores
NW = NC * NS
ROWS_PER_W = B // NW
CH = 32
N_CHUNK = ROWS_PER_W // CH


def _sc_body(b0, b1i, b2i, mtT, gtT, m_emb, g_emb,
             org, gg, mem, cm, cg,
             idx0, idx1, rows0, rows1, ids0, ids1, cnt0, cnt1,
             gsem0, gsem1, osem0, osem1):
    wid = lax.axis_index("s") * NC + lax.axis_index("c")
    base = wid * ROWS_PER_W

    plan = ((b0, m_emb, org), (b1i, g_emb, gg), (b2i, m_emb, mem))
    bufs = ((idx0, rows0, gsem0, osem0), (idx1, rows1, gsem1, osem1))
    gather_h = [None, None]
    out_h = [None, None]
    for i, (src, tab, out) in enumerate(plan):
        idxb, rowsb, gs, os_ = bufs[i % 2]
        if out_h[i % 2] is not None:
            out_h[i % 2].wait()
        pltpu.sync_copy(src.at[pl.ds(base, ROWS_PER_W)], idxb)
        gather_h[i % 2] = pltpu.async_copy(tab.at[idxb], rowsb, gs)
        if i >= 1:
            p = (i - 1) % 2
            gather_h[p].wait()
            out_h[p] = pltpu.async_copy(
                bufs[p][1], plan[i - 1][2].at[pl.ds(base, ROWS_PER_W)],
                bufs[p][3])
    gather_h[2 % 2].wait()
    out_h[0] = pltpu.async_copy(rows0, mem.at[pl.ds(base, ROWS_PER_W)], osem0)
    out_h[0].wait()
    out_h[1].wait()

    zeros16 = jnp.zeros((16,), jnp.float32)
    ones16 = jnp.ones((16,), jnp.float32)
    lane = lax.iota(jnp.int32, 16)
    full_mask = lane < 16

    def make_scatter(idsb, cntb, c, vals):
        def body(h, _):
            for g in range(CH // 16):
                cols = idsb[h, pl.ds(c * CH + g * 16, 16)]
                rows = lane + (g * 16)
                if vals is None:
                    plsc.store_scatter(cntb, [rows, cols], zeros16,
                                       mask=full_mask)
                else:
                    plsc.addupdate_scatter(cntb, [rows, cols], vals,
                                           mask=full_mask)
            return 0
        return body

    def zero_row(cntb):
        def body(r, _):
            for j in range(NT_PAD // 16):
                cntb[r, pl.ds(j * 16, 16)] = zeros16
            return 0
        return body

    lax.fori_loop(0, CH, zero_row(cnt0), 0)
    lax.fori_loop(0, CH, zero_row(cnt1), 0)
    pltpu.sync_copy(mtT.at[:, pl.ds(base, ROWS_PER_W)], ids0)
    pltpu.sync_copy(gtT.at[:, pl.ds(base, ROWS_PER_W)], ids1)

    work = [(ids0, cm, c) for c in range(N_CHUNK)] + \
           [(ids1, cg, c) for c in range(N_CHUNK)]
    cbufs = ((cnt0, gsem0), (cnt1, gsem1))
    cout_h = [None, None]
    prev = [None, None]
    for k, (idsb, outf, c) in enumerate(work):
        bidx = k % 2
        cntb, osem_b = cbufs[bidx]
        if cout_h[bidx] is not None:
            cout_h[bidx].wait()
            pids, pc = prev[bidx]
            lax.fori_loop(0, HIST, make_scatter(pids, cntb, pc, None), 0)
        lax.fori_loop(0, HIST, make_scatter(idsb, cntb, c, ones16), 0)
        cout_h[bidx] = pltpu.async_copy(
            cntb, outf.at[pl.ds(base + c * CH, CH)], osem_b)
        prev[bidx] = (idsb, c)
    cout_h[0].wait()
    cout_h[1].wait()


@jax.jit
def _sc_call(b0, b1i, b2i, mtT, gtT, m_emb, g_emb):
    kern = pl.kernel(
        _sc_body,
        out_type=[
            jax.ShapeDtypeStruct((B, E), jnp.float32),
            jax.ShapeDtypeStruct((B, E), jnp.float32),
            jax.ShapeDtypeStruct((B, E), jnp.float32),
            jax.ShapeDtypeStruct((B, NT_PAD), jnp.float32),
            jax.ShapeDtypeStruct((B, NT_PAD), jnp.float32),
        ],
        mesh=plsc.VectorSubcoreMesh(core_axis_name="c", subcore_axis_name="s"),
        compiler_params=pltpu.CompilerParams(needs_layout_passes=False),
        scratch_types=[
            pltpu.VMEM((ROWS_PER_W,), jnp.int32),
            pltpu.VMEM((ROWS_PER_W,), jnp.int32),
            pltpu.VMEM((ROWS_PER_W, E), jnp.float32),
            pltpu.VMEM((ROWS_PER_W, E), jnp.float32),
            pltpu.VMEM((HIST, ROWS_PER_W), jnp.int32),
            pltpu.VMEM((HIST, ROWS_PER_W), jnp.int32),
            pltpu.VMEM((CH, NT_PAD), jnp.float32),
            pltpu.VMEM((CH, NT_PAD), jnp.float32),
            pltpu.SemaphoreType.DMA,
            pltpu.SemaphoreType.DMA,
            pltpu.SemaphoreType.DMA,
            pltpu.SemaphoreType.DMA,
        ],
    )
    return kern(b0, b1i, b2i, mtT, gtT, m_emb, g_emb)


def _tc_body(org_ref, gg_ref, mem_ref, cm_ref, cg_ref, temb_ref,
             w1_ref, b1_ref, w2_ref, b2_ref, w3tp_ref, b3c_ref, out_ref):
    te = temb_ref[...]
    pm = jnp.dot(cm_ref[...], te, preferred_element_type=jnp.float32) / 50.0
    pg = jnp.dot(cg_ref[...], te, preferred_element_type=jnp.float32) / 50.0
    x = jnp.concatenate(
        [org_ref[...], gg_ref[...], mem_ref[...], pm, pg], axis=1)
    h = jnp.dot(x, w1_ref[...], preferred_element_type=jnp.float32) + b1_ref[...]
    h = jnp.maximum(h, 0.0)
    h = jnp.dot(h, w2_ref[...], preferred_element_type=jnp.float32) + b2_ref[...]
    h = jnp.maximum(h, 0.0)
    outT = lax.dot_general(w3tp_ref[...], h, (((1,), (1,)), ((), ())),
                           preferred_element_type=jnp.float32)
    out_ref[...] = outT + b3c_ref[...]


def _tc_call(org, gg, mem, cm, cg, t_emb, W1, b1, W2, b2, W3tp, b3c):
    BLK = 512
    grid = (B // BLK,)

    def row_spec(width):
        return pl.BlockSpec((BLK, width), lambda i: (i, 0))

    def full_spec(shape):
        return pl.BlockSpec(shape, lambda i: tuple(0 for _ in shape))

    return pl.pallas_call(
        _tc_body,
        grid=grid,
        in_specs=[
            row_spec(E), row_spec(E), row_spec(E),
            row_spec(NT_PAD), row_spec(NT_PAD),
            full_spec(t_emb.shape),
            full_spec(W1.shape), full_spec((1, H)),
            full_spec(W2.shape), full_spec((1, H)),
            full_spec(W3tp.shape), full_spec((N_OUT, 1)),
        ],
        out_specs=pl.BlockSpec((N_OUT, BLK), lambda i: (0, i)),
        out_shape=jax.ShapeDtypeStruct((N_OUT, B), jnp.float32),
        compiler_params=pltpu.CompilerParams(
            dimension_semantics=("parallel",),
        ),
    )(org, gg, mem, cm, cg, t_emb, W1, b1, W2, b2, W3tp, b3c)


def kernel(batch, m_topic, g_topic, m_emb, g_emb, t_emb, W1, b1, W2, b2, W3, b3):
    b0 = batch[:, 0].astype(jnp.int32)
    b1i = batch[:, 1].astype(jnp.int32)
    b2i = batch[:, 2].astype(jnp.int32)
    mtT = m_topic.T.astype(jnp.int32)
    gtT = g_topic.T.astype(jnp.int32)
    t_emb_pad = jnp.pad(t_emb, ((0, NT_PAD - N_TOPIC), (0, 0)))

    org, gg, mem, cm, cg = _sc_call(b0, b1i, b2i, mtT, gtT, m_emb, g_emb)

    outT = _tc_call(org, gg, mem, cm, cg, t_emb_pad,
                    W1, b1.reshape(1, H), W2, b2.reshape(1, H),
                    W3.T, b3.reshape(-1, 1))
    return (outT.T,)


# speedup vs baseline: 24.4284x; 1.0353x over previous
import functools

import jax
import jax.numpy as jnp
from jax import lax
from jax.experimental import pallas as pl
from jax.experimental.pallas import tpu as pltpu
from jax.experimental.pallas import tpu_sc as plsc

B = 4096
E = 128
H = 1024
N_TOPIC = 1000
NT_PAD = 1024
N_OUT = 1000
HIST = 50

_info = plsc.get_sparse_core_info()
NC, NS = _info.num_cores, _info.num_subcores
NW = NC * NS
ROWS_PER_W = B // NW
CH = 32
N_CHUNK = ROWS_PER_W // CH


def _sc_body(b0, b1i, b2i, mtT, gtT, m_emb, g_emb,
             org, gg, mem, cc,
             idx0, idx1, rows0, rows1, ids0, ids1, cnt0, cnt1,
             gsem0, gsem1, osem0, osem1):
    wid = lax.axis_index("s") * NC + lax.axis_index("c")
    base = wid * ROWS_PER_W

    plan = ((b0, m_emb, org), (b1i, g_emb, gg), (b2i, m_emb, mem))
    bufs = ((idx0, rows0, gsem0, osem0), (idx1, rows1, gsem1, osem1))
    gather_h = [None, None]
    out_h = [None, None]
    for i, (src, tab, out) in enumerate(plan):
        idxb, rowsb, gs, os_ = bufs[i % 2]
        if out_h[i % 2] is not None:
            out_h[i % 2].wait()
        pltpu.sync_copy(src.at[pl.ds(base, ROWS_PER_W)], idxb)
        gather_h[i % 2] = pltpu.async_copy(tab.at[idxb], rowsb, gs)
        if i >= 1:
            p = (i - 1) % 2
            gather_h[p].wait()
            out_h[p] = pltpu.async_copy(
                bufs[p][1], plan[i - 1][2].at[pl.ds(base, ROWS_PER_W)],
                bufs[p][3])
    gather_h[2 % 2].wait()
    out_h[0] = pltpu.async_copy(rows0, mem.at[pl.ds(base, ROWS_PER_W)], osem0)
    out_h[0].wait()
    out_h[1].wait()

    zeros16 = jnp.zeros((16,), jnp.float32)
    ones16 = jnp.ones((16,), jnp.float32)
    c256 = jnp.full((16,), 256.0, jnp.float32)
    lane = lax.iota(jnp.int32, 16)
    full_mask = lane < 16

    def make_scatter(idsb, cntb, c, vals):
        def body(h, _):
            for g in range(CH // 16):
                cols = idsb[h, pl.ds(c * CH + g * 16, 16)]
                rows = lane + (g * 16)
                if vals is None:
                    plsc.store_scatter(cntb, [rows, cols], zeros16,
                                       mask=full_mask)
                else:
                    plsc.addupdate_scatter(cntb, [rows, cols], vals,
                                           mask=full_mask)
            return 0
        return body

    def zero_row(cntb):
        def body(r, _):
            for j in range(NT_PAD // 16):
                cntb[r, pl.ds(j * 16, 16)] = zeros16
            return 0
        return body

    lax.fori_loop(0, CH, zero_row(cnt0), 0)
    lax.fori_loop(0, CH, zero_row(cnt1), 0)
    pltpu.sync_copy(mtT.at[:, pl.ds(base, ROWS_PER_W)], ids0)
    pltpu.sync_copy(gtT.at[:, pl.ds(base, ROWS_PER_W)], ids1)

    cbufs = ((cnt0, gsem0), (cnt1, gsem1))
    cout_h = [None, None]
    prev = [None, None]
    for c in range(N_CHUNK):
        bidx = c % 2
        cntb, osem_b = cbufs[bidx]
        if cout_h[bidx] is not None:
            cout_h[bidx].wait()
            pc = prev[bidx]
            lax.fori_loop(0, HIST, make_scatter(ids0, cntb, pc, None), 0)
            lax.fori_loop(0, HIST, make_scatter(ids1, cntb, pc, None), 0)
        lax.fori_loop(0, HIST, make_scatter(ids0, cntb, c, ones16), 0)
        lax.fori_loop(0, HIST, make_scatter(ids1, cntb, c, c256), 0)
        cout_h[bidx] = pltpu.async_copy(
            cntb, cc.at[pl.ds(base + c * CH, CH)], osem_b)
        prev[bidx] = c
    cout_h[0].wait()
    cout_h[1].wait()


@jax.jit
def _sc_call(b0, b1i, b2i, mtT, gtT, m_emb, g_emb):
    kern = pl.kernel(
        _sc_body,
        out_type=[
            jax.ShapeDtypeStruct((B, E), jnp.float32),
            jax.ShapeDtypeStruct((B, E), jnp.float32),
            jax.ShapeDtypeStruct((B, E), jnp.float32),
            jax.ShapeDtypeStruct((B, NT_PAD), jnp.float32),
        ],
        mesh=plsc.VectorSubcoreMesh(core_axis_name="c", subcore_axis_name="s"),
        compiler_params=pltpu.CompilerParams(needs_layout_passes=False),
        scratch_types=[
            pltpu.VMEM((ROWS_PER_W,), jnp.int32),
            pltpu.VMEM((ROWS_PER_W,), jnp.int32),
            pltpu.VMEM((ROWS_PER_W, E), jnp.float32),
            pltpu.VMEM((ROWS_PER_W, E), jnp.float32),
            pltpu.VMEM((HIST, ROWS_PER_W), jnp.int32),
            pltpu.VMEM((HIST, ROWS_PER_W), jnp.int32),
            pltpu.VMEM((CH, NT_PAD), jnp.float32),
            pltpu.VMEM((CH, NT_PAD), jnp.float32),
            pltpu.SemaphoreType.DMA,
            pltpu.SemaphoreType.DMA,
            pltpu.SemaphoreType.DMA,
            pltpu.SemaphoreType.DMA,
        ],
    )
    return kern(b0, b1i, b2i, mtT, gtT, m_emb, g_emb)


def _tc_body(org_ref, gg_ref, mem_ref, cc_ref, temb_ref,
             w1_ref, b1_ref, w2_ref, b2_ref, w3tp_ref, b3c_ref, out_ref):
    bf = jnp.bfloat16
    te = temb_ref[...]
    c = cc_ref[...]
    cgf = jnp.floor(c * (1.0 / 256.0))
    cmf = c - cgf * 256.0
    pm = jnp.dot(cmf.astype(bf), te,
                 preferred_element_type=jnp.float32) / 50.0
    pg = jnp.dot(cgf.astype(bf), te,
                 preferred_element_type=jnp.float32) / 50.0
    x = jnp.concatenate(
        [org_ref[...].astype(bf), gg_ref[...].astype(bf),
         mem_ref[...].astype(bf), pm.astype(bf), pg.astype(bf)], axis=1)
    h = jnp.dot(x, w1_ref[...], preferred_element_type=jnp.float32) + b1_ref[...]
    h = jnp.maximum(h, 0.0).astype(bf)
    h = jnp.dot(h, w2_ref[...], preferred_element_type=jnp.float32) + b2_ref[...]
    h = jnp.maximum(h, 0.0).astype(bf)
    outT = lax.dot_general(w3tp_ref[...], h, (((1,), (1,)), ((), ())),
                           preferred_element_type=jnp.float32)
    out_ref[...] = outT + b3c_ref[...]


def _tc_call(org, gg, mem, cc, t_emb, W1, b1, W2, b2, W3tp, b3c):
    BLK = 512
    grid = (B // BLK,)

    def row_spec(width):
        return pl.BlockSpec((BLK, width), lambda i: (i, 0))

    def full_spec(shape):
        return pl.BlockSpec(shape, lambda i: tuple(0 for _ in shape))

    return pl.pallas_call(
        _tc_body,
        grid=grid,
        in_specs=[
            row_spec(E), row_spec(E), row_spec(E),
            row_spec(NT_PAD),
            full_spec(t_emb.shape),
            full_spec(W1.shape), full_spec((1, H)),
            full_spec(W2.shape), full_spec((1, H)),
            full_spec(W3tp.shape), full_spec((N_OUT, 1)),
        ],
        out_specs=pl.BlockSpec((N_OUT, BLK), lambda i: (0, i)),
        out_shape=jax.ShapeDtypeStruct((N_OUT, B), jnp.float32),
        compiler_params=pltpu.CompilerParams(
            dimension_semantics=("parallel",),
        ),
    )(org, gg, mem, cc, t_emb, W1, b1, W2, b2, W3tp, b3c)


def kernel(batch, m_topic, g_topic, m_emb, g_emb, t_emb, W1, b1, W2, b2, W3, b3):
    b0 = batch[:, 0].astype(jnp.int32)
    b1i = batch[:, 1].astype(jnp.int32)
    b2i = batch[:, 2].astype(jnp.int32)
    mtT = m_topic.T.astype(jnp.int32)
    gtT = g_topic.T.astype(jnp.int32)
    t_emb_pad = jnp.pad(t_emb, ((0, NT_PAD - N_TOPIC), (0, 0)))

    org, gg, mem, cc = _sc_call(b0, b1i, b2i, mtT, gtT, m_emb, g_emb)

    bf = jnp.bfloat16
    outT = _tc_call(org, gg, mem, cc, t_emb_pad.astype(bf),
                    W1.astype(bf), b1.reshape(1, H),
                    W2.astype(bf), b2.reshape(1, H),
                    W3.T.astype(bf), b3.reshape(-1, 1))
    return (outT.T,)
